# baseline (speedup 1.0000x reference)
import numpy as np

import jax
import jax.numpy as jnp
from jax.experimental import pallas as pl
from jax.experimental.pallas import tpu as pltpu

HW_IN = 32
NPAD = 128
NFUSE = 2 * NPAD
P1 = 14
P2 = 5
NCLASS = 102
KS = 5
IC1 = 3
NQ = 24
K1 = 2 * IC1 * NPAD
N1 = 4 * NFUSE
KCAT = KS * NPAD


def _lenet_kernel(x_ref, p_ref,
                  w1_ref, b1_ref, w2_ref, b2_ref,
                  wf1_ref, bf1_ref, wf2_ref, bf2_ref, wf3_ref, bf3_ref,
                  o_ref):
    f32 = jnp.float32
    bf16 = jnp.bfloat16
    bt = o_ref.shape[0]

    a_band = jnp.dot(p_ref[...], w1_ref[...].astype(bf16),
                     preferred_element_type=f32).astype(bf16)
    w1c = jnp.concatenate(
        [a_band if m == 0 else
         jnp.concatenate([jnp.zeros((HW_IN * m, NFUSE), bf16),
                          a_band[:K1 - HW_IN * m]], axis=0)
         for m in range(4)], axis=1)

    xq = jnp.transpose(x_ref[...], (1, 0, 2)).astype(bf16)

    xcat = jnp.concatenate(
        [xq[8 * c + qr:8 * c + qr + 7] for c in range(IC1) for qr in (0, 1)],
        axis=-1)
    res = jnp.dot(xcat.reshape(7 * bt, K1), w1c,
                  preferred_element_type=f32)
    res = res.reshape(7, bt, N1)
    ev = jnp.maximum(
        jnp.maximum(res[..., 0 * NPAD:1 * NPAD], res[..., 1 * NPAD:2 * NPAD]),
        jnp.maximum(res[..., 2 * NPAD:3 * NPAD], res[..., 3 * NPAD:4 * NPAD]))
    od = jnp.maximum(
        jnp.maximum(res[..., 4 * NPAD:5 * NPAD], res[..., 5 * NPAD:6 * NPAD]),
        jnp.maximum(res[..., 6 * NPAD:7 * NPAD], res[..., 7 * NPAD:8 * NPAD]))
    m1 = jnp.stack([ev, od], axis=1).reshape(P1, bt, NPAD)
    h1 = jnp.maximum(m1 + b1_ref[...], 0.0).astype(bf16)

    hcat = jnp.concatenate([h1[k:k + 2 * P2] for k in range(KS)],
                           axis=-1)
    res2 = jnp.dot(hcat.reshape(2 * P2 * bt, KCAT), w2_ref[...].astype(bf16),
                   preferred_element_type=f32)
    res2 = res2.reshape(P2, 2, bt, NFUSE)
    m2 = jnp.maximum(jnp.maximum(res2[:, 0, :, :NPAD], res2[:, 0, :, NPAD:]),
                     jnp.maximum(res2[:, 1, :, :NPAD], res2[:, 1, :, NPAD:]))
    h2 = jnp.maximum(m2 + b2_ref[...], 0.0).astype(bf16)

    hf = jnp.concatenate([h2[r] for r in range(P2)], axis=-1)
    a = jnp.dot(hf, wf1_ref[...].astype(bf16), preferred_element_type=f32)
    a = jnp.maximum(a + bf1_ref[...], 0.0).astype(bf16)
    a = jnp.maximum(jnp.dot(a, wf2_ref[...].astype(bf16),
                            preferred_element_type=f32)
                    + bf2_ref[...], 0.0).astype(bf16)
    res3 = (jnp.dot(a, wf3_ref[...].astype(bf16), preferred_element_type=f32)
            + bf3_ref[...])
    o_ref[...] = res3[:, :NCLASS]


def _band_perm():
    p = np.zeros((K1, KS * IC1 * HW_IN), np.float32)
    for c in range(IC1):
        for t in range(KS):
            for w in range(HW_IN):
                p[c * 256 + t * HW_IN + w, t * 96 + w * IC1 + c] = 1.0
    return p


_P_BAND = _band_perm()


def kernel(x_nchw, w1, b1, w2, b2, wf1, bf1, wf2, bf2, wf3, bf3):
    B = x_nchw.shape[0]
    bf16 = jnp.bfloat16
    Bt = B if B <= 512 else 512
    nblk = -(-B // Bt)
    Bp = nblk * Bt

    x = x_nchw.reshape(B, NQ, NPAD)
    if Bp != B:
        x = jnp.pad(x, ((0, Bp - B), (0, 0), (0, 0)))

    pband = jnp.asarray(_P_BAND, jnp.bfloat16)
    weights = (w1.reshape(KS * IC1 * HW_IN, NFUSE), b1,
               w2.reshape(KCAT, NFUSE), b2,
               wf1.reshape(KCAT, NPAD), bf1, wf2, bf2, wf3, bf3)

    def _const_spec(a):
        return pl.BlockSpec(a.shape, lambda b: (0,) * a.ndim)

    in_specs = [pl.BlockSpec((Bt, NQ, NPAD), lambda b: (b, 0, 0)),
                _const_spec(pband)]
    in_specs += [_const_spec(a) for a in weights]

    out = pl.pallas_call(
        _lenet_kernel,
        out_shape=jax.ShapeDtypeStruct((Bp, NCLASS), jnp.float32),
        grid=(nblk,),
        in_specs=in_specs,
        out_specs=pl.BlockSpec((Bt, NCLASS), lambda b: (b, 0)),
        compiler_params=pltpu.CompilerParams(
            dimension_semantics=("parallel",)),
    )(x, pband, *weights)
    return out if Bp == B else out[:B]


# speedup vs baseline: 1.0071x; 1.0071x over previous
import jax
import jax.numpy as jnp
from jax.experimental import pallas as pl
from jax.experimental.pallas import tpu as pltpu

HW_IN = 32
NPAD = 128
NFUSE = 2 * NPAD
P1 = 14
P2 = 5
NCLASS = 102
KS = 5
IC1 = 3
NQ = 24
K1 = 2 * IC1 * NPAD
N1 = 4 * NFUSE
KCAT = KS * NPAD


def _lenet_kernel(x_ref,
                  w1_ref, b1_ref, w2_ref, b2_ref,
                  wf1_ref, bf1_ref, wf2_ref, bf2_ref, wf3_ref, bf3_ref,
                  o_ref):
    f32 = jnp.float32
    bf16 = jnp.bfloat16
    bt = o_ref.shape[0]

    w1c = w1_ref[...]

    xq = jnp.transpose(x_ref[...], (1, 0, 2)).astype(bf16)

    xcat = jnp.concatenate(
        [xq[8 * c + qr:8 * c + qr + 7] for c in range(IC1) for qr in (0, 1)],
        axis=-1)
    res = jnp.dot(xcat.reshape(7 * bt, K1), w1c,
                  preferred_element_type=f32)
    res = res.reshape(7, bt, N1)
    ev = jnp.maximum(
        jnp.maximum(res[..., 0 * NPAD:1 * NPAD], res[..., 1 * NPAD:2 * NPAD]),
        jnp.maximum(res[..., 2 * NPAD:3 * NPAD], res[..., 3 * NPAD:4 * NPAD]))
    od = jnp.maximum(
        jnp.maximum(res[..., 4 * NPAD:5 * NPAD], res[..., 5 * NPAD:6 * NPAD]),
        jnp.maximum(res[..., 6 * NPAD:7 * NPAD], res[..., 7 * NPAD:8 * NPAD]))
    m1 = jnp.stack([ev, od], axis=1).reshape(P1, bt, NPAD)
    h1 = jnp.maximum(m1 + b1_ref[...], 0.0).astype(bf16)

    hcat = jnp.concatenate([h1[k:k + 2 * P2] for k in range(KS)],
                           axis=-1)
    res2 = jnp.dot(hcat.reshape(2 * P2 * bt, KCAT), w2_ref[...].astype(bf16),
                   preferred_element_type=f32)
    res2 = res2.reshape(P2, 2, bt, NFUSE)
    m2 = jnp.maximum(jnp.maximum(res2[:, 0, :, :NPAD], res2[:, 0, :, NPAD:]),
                     jnp.maximum(res2[:, 1, :, :NPAD], res2[:, 1, :, NPAD:]))
    h2 = jnp.maximum(m2 + b2_ref[...], 0.0).astype(bf16)

    hf = jnp.concatenate([h2[r] for r in range(P2)], axis=-1)
    a = jnp.dot(hf, wf1_ref[...].astype(bf16), preferred_element_type=f32)
    a = jnp.maximum(a + bf1_ref[...], 0.0).astype(bf16)
    a = jnp.maximum(jnp.dot(a, wf2_ref[...].astype(bf16),
                            preferred_element_type=f32)
                    + bf2_ref[...], 0.0).astype(bf16)
    res3 = (jnp.dot(a, wf3_ref[...].astype(bf16), preferred_element_type=f32)
            + bf3_ref[...])
    o_ref[...] = res3[:, :NCLASS]


def _band_conv1(w1):
    a = w1.reshape(KS, HW_IN, IC1, NFUSE).transpose(2, 0, 1, 3)
    a = jnp.pad(a, ((0, 0), (0, 3), (0, 0), (0, 0)))
    a = a.reshape(K1, NFUSE)
    return jnp.concatenate(
        [jnp.pad(a[:K1 - HW_IN * m], ((HW_IN * m, 0), (0, 0)))
         for m in range(4)], axis=1)


def kernel(x_nchw, w1, b1, w2, b2, wf1, bf1, wf2, bf2, wf3, bf3):
    B = x_nchw.shape[0]
    bf16 = jnp.bfloat16
    Bt = B if B <= 512 else 512
    nblk = -(-B // Bt)
    Bp = nblk * Bt

    x = x_nchw.reshape(B, NQ, NPAD)
    if Bp != B:
        x = jnp.pad(x, ((0, Bp - B), (0, 0), (0, 0)))

    weights = (_band_conv1(w1).astype(bf16), b1,
               w2.reshape(KCAT, NFUSE), b2,
               wf1.reshape(KCAT, NPAD), bf1, wf2, bf2, wf3, bf3)

    def _const_spec(a):
        return pl.BlockSpec(a.shape, lambda b: (0,) * a.ndim)

    in_specs = [pl.BlockSpec((Bt, NQ, NPAD), lambda b: (b, 0, 0))]
    in_specs += [_const_spec(a) for a in weights]

    out = pl.pallas_call(
        _lenet_kernel,
        out_shape=jax.ShapeDtypeStruct((Bp, NCLASS), jnp.float32),
        grid=(nblk,),
        in_specs=in_specs,
        out_specs=pl.BlockSpec((Bt, NCLASS), lambda b: (b, 0)),
        compiler_params=pltpu.CompilerParams(
            dimension_semantics=("parallel",)),
    )(x, *weights)
    return out if Bp == B else out[:B]
